# Initial kernel scaffold; baseline (speedup 1.0000x reference)
import jax
import jax.numpy as jnp
from jax.experimental import pallas as pl


def kernel(seq, table):
    raise NotImplementedError("write your pallas kernel here")


# speedup vs baseline: 4.1798x; 4.1798x over previous
import jax
import jax.numpy as jnp
from jax.experimental import pallas as pl
from jax.experimental.pallas import tpu as pltpu

_B = 4096
_L = 200
_D = 128
_V = 6
_BB = 16


def _body(seq_ref, table_ref, out_ref):
    s = seq_ref[...]
    t = table_ref[...]
    m = s[:, None, :]
    acc = jnp.zeros((_BB, _D, _L), jnp.float32)
    for v in range(1, _V):
        acc = jnp.where(m == v, t[v][None, :, None], acc)
    out_ref[...] = acc


def kernel(seq, table):
    seq = seq.astype(jnp.int32)
    grid = (_B // _BB,)
    return pl.pallas_call(
        _body,
        grid=grid,
        in_specs=[
            pl.BlockSpec((_BB, _L), lambda i: (i, 0)),
            pl.BlockSpec((_V, _D), lambda i: (0, 0)),
        ],
        out_specs=pl.BlockSpec((_BB, _D, _L), lambda i: (i, 0, 0)),
        out_shape=jax.ShapeDtypeStruct((_B, _D, _L), jnp.float32),
    )(seq, table)


# speedup vs baseline: 4.3989x; 1.0524x over previous
import jax
import jax.numpy as jnp
from jax.experimental import pallas as pl
from jax.experimental.pallas import tpu as pltpu

_B = 4096
_L = 200
_D = 128
_V = 6
_BB = 16


_VP = 8


def _body(seq_ref, tableT_ref, out_ref):
    tt = tableT_ref[...]
    viota = jax.lax.broadcasted_iota(jnp.int32, (_VP, _L), 0)
    for i in range(_BB):
        s = seq_ref[i, :]
        oh = (s[None, :] == viota).astype(jnp.float32)
        out_ref[i, :, :] = jnp.dot(tt, oh, preferred_element_type=jnp.float32)


def kernel(seq, table):
    seq = seq.astype(jnp.int32)
    tableT = jnp.zeros((_D, _VP), jnp.float32).at[:, :_V].set(table.T)
    grid = (_B // _BB,)
    return pl.pallas_call(
        _body,
        grid=grid,
        in_specs=[
            pl.BlockSpec((_BB, _L), lambda i: (i, 0)),
            pl.BlockSpec((_D, _VP), lambda i: (0, 0)),
        ],
        out_specs=pl.BlockSpec((_BB, _D, _L), lambda i: (i, 0, 0)),
        out_shape=jax.ShapeDtypeStruct((_B, _D, _L), jnp.float32),
    )(seq, tableT)


# speedup vs baseline: 4.9975x; 1.1361x over previous
import jax
import jax.numpy as jnp
from jax.experimental import pallas as pl
from jax.experimental.pallas import tpu as pltpu

_B = 4096
_L = 200
_D = 128
_V = 6
_BB = 64


_VP = 8


def _body(seq_ref, tableT_ref, out_ref):
    tt = tableT_ref[...]
    viota = jax.lax.broadcasted_iota(jnp.int32, (_VP, _L), 0)
    for i in range(_BB):
        s = seq_ref[i, :]
        oh = (s[None, :] == viota).astype(jnp.float32)
        out_ref[i, :, :] = jnp.dot(tt, oh, preferred_element_type=jnp.float32)


def kernel(seq, table):
    seq = seq.astype(jnp.int32)
    tableT = jnp.zeros((_D, _VP), jnp.float32).at[:, :_V].set(table.T)
    grid = (_B // _BB,)
    return pl.pallas_call(
        _body,
        grid=grid,
        in_specs=[
            pl.BlockSpec((_BB, _L), lambda i: (i, 0)),
            pl.BlockSpec((_D, _VP), lambda i: (0, 0)),
        ],
        out_specs=pl.BlockSpec((_BB, _D, _L), lambda i: (i, 0, 0)),
        out_shape=jax.ShapeDtypeStruct((_B, _D, _L), jnp.float32),
    )(seq, tableT)
